# baseline (speedup 1.0000x reference)
import functools

import jax
import jax.numpy as jnp
from jax import lax
from jax.experimental import pallas as pl
from jax.experimental.pallas import tpu as pltpu
from jax.experimental.pallas import tpu_sc as plsc

N_EXP = 8
HID = 1024
TOK = 4096
BLK = 512
NUM_BLOCKS = TOK // BLK + N_EXP
PADDED = NUM_BLOCKS * BLK

NC = 2
NS = 16
NW = NC * NS


_ROWS = 32
_LANES = TOK // _ROWS
_NA_LANE = 120


def _route_body(e_ref, dest_ref, meta_ref):
    e = e_ref[...]
    li = lax.broadcasted_iota(jnp.int32, (_LANES, _LANES), 0)
    lj = lax.broadcasted_iota(jnp.int32, (_LANES, _LANES), 1)
    tri_incl = (li <= lj).astype(jnp.bfloat16)
    ri = lax.broadcasted_iota(jnp.int32, (_ROWS, _ROWS), 0)
    rj = lax.broadcasted_iota(jnp.int32, (_ROWS, _ROWS), 1)
    tri_strict = (rj < ri).astype(jnp.bfloat16)
    ones_col = jnp.ones((_LANES, 1), jnp.bfloat16)

    dest = jnp.zeros((_ROWS, _LANES), jnp.float32)
    offs = 0.0
    cumblk = []
    for k in range(N_EXP):
        m = (e == k).astype(jnp.bfloat16)
        pfx = lax.dot_general(
            m, tri_incl, (((1,), (0,)), ((), ())),
            preferred_element_type=jnp.float32,
        )
        rowsum = lax.dot_general(
            m, ones_col, (((1,), (0,)), ((), ())),
            preferred_element_type=jnp.float32,
        )
        rowpfx = lax.dot_general(
            tri_strict, rowsum.astype(jnp.bfloat16), (((1,), (0,)), ((), ())),
            preferred_element_type=jnp.float32,
        )
        g = pfx + rowpfx
        count = jnp.sum(rowsum)
        padded = jnp.floor((count + (BLK - 1)) * (1.0 / BLK)) * BLK
        dest = dest + m.astype(jnp.float32) * (offs - 1.0 + g)
        offs = offs + padded
        cumblk.append(offs * (1.0 / BLK))
    dest_ref[...] = dest.astype(jnp.int32)

    lane = lax.broadcasted_iota(jnp.int32, (8, _LANES), 1).astype(jnp.float32)
    be = jnp.zeros((8, _LANES), jnp.float32)
    for k in range(N_EXP):
        be = be + (lane >= cumblk[k]).astype(jnp.float32)
    be = jnp.minimum(be, float(N_EXP - 1))
    be = jnp.where(lane == float(_NA_LANE), cumblk[N_EXP - 1], be)
    meta_ref[...] = be.astype(jnp.int32)


def _route_metadata(chosen_ops):
    e32 = chosen_ops.astype(jnp.int32).reshape(_ROWS, _LANES)
    dest, meta = pl.pallas_call(
        _route_body,
        out_shape=(
            jax.ShapeDtypeStruct((_ROWS, _LANES), jnp.int32),
            jax.ShapeDtypeStruct((8, _LANES), jnp.int32),
        ),
    )(e32)
    return dest, meta


def _sc_scatter_rows(x, dest):
    bpw = TOK // NW
    chunk = 32
    n_chunks = bpw // chunk
    dest3 = dest.reshape(NW, n_chunks, chunk)
    mesh = plsc.VectorSubcoreMesh(core_axis_name="c", subcore_axis_name="s")

    @functools.partial(
        pl.kernel,
        mesh=mesh,
        out_type=jax.ShapeDtypeStruct((PADDED, HID), jnp.float32),
        scratch_types=[
            pltpu.VMEM((n_chunks, chunk), jnp.int32),
            pltpu.VMEM((chunk, HID), jnp.float32),
            pltpu.VMEM((chunk, HID), jnp.float32),
            pltpu.SemaphoreType.DMA,
            pltpu.SemaphoreType.DMA,
        ],
    )
    def scatter_kernel(x_hbm, idx_hbm, out_hbm, idx_v, rows_a, rows_b, sem_a, sem_b):
        wid = lax.axis_index("s") * NC + lax.axis_index("c")
        base = wid * bpw
        pltpu.sync_copy(idx_hbm.at[wid], idx_v)
        bufs = (rows_a, rows_b)
        sems = (sem_a, sem_b)
        copies = [None, None]
        for c in range(n_chunks):
            if c >= 2:
                copies[c % 2].wait()
            pltpu.sync_copy(x_hbm.at[pl.ds(base + c * chunk, chunk)], bufs[c % 2])
            copies[c % 2] = pltpu.async_copy(
                bufs[c % 2], out_hbm.at[idx_v.at[c]], sems[c % 2]
            )
        copies[(n_chunks - 2) % 2].wait()
        copies[(n_chunks - 1) % 2].wait()

    return scatter_kernel(x, dest3)


def _sc_gather_rows(table, idx, n_rows):
    bpw = n_rows // NW
    chunk = 32
    n_chunks = bpw // chunk
    mesh = plsc.VectorSubcoreMesh(core_axis_name="c", subcore_axis_name="s")

    @functools.partial(
        pl.kernel,
        mesh=mesh,
        out_type=jax.ShapeDtypeStruct((n_rows, HID), jnp.float32),
        scratch_types=[
            pltpu.VMEM((bpw,), jnp.int32),
            pltpu.VMEM((chunk, HID), jnp.float32),
            pltpu.VMEM((chunk, HID), jnp.float32),
            pltpu.SemaphoreType.DMA,
            pltpu.SemaphoreType.DMA,
        ],
    )
    def gather_kernel(table_hbm, idx_hbm, out_hbm, idx_v, rows_a, rows_b, sem_a, sem_b):
        wid = lax.axis_index("s") * NC + lax.axis_index("c")
        base = wid * bpw
        pltpu.sync_copy(idx_hbm.at[pl.ds(base, bpw)], idx_v)
        bufs = (rows_a, rows_b)
        sems = (sem_a, sem_b)
        copies = [None, None]
        for c in range(n_chunks):
            copies[c % 2] = pltpu.async_copy(
                table_hbm.at[idx_v.at[pl.ds(c * chunk, chunk)]],
                bufs[c % 2],
                sems[c % 2],
            )
            if c >= 1:
                copies[(c - 1) % 2].wait()
                pltpu.sync_copy(
                    bufs[(c - 1) % 2],
                    out_hbm.at[pl.ds(base + (c - 1) * chunk, chunk)],
                )
        copies[(n_chunks - 1) % 2].wait()
        pltpu.sync_copy(
            bufs[(n_chunks - 1) % 2],
            out_hbm.at[pl.ds(base + (n_chunks - 1) * chunk, chunk)],
        )

    return gather_kernel(table, idx)


def _tc_mlp_body(meta_ref, xs_ref, w1_ref, b1_ref, w2_ref, b2_ref, out_ref):
    i = pl.program_id(0)

    @pl.when(i < meta_ref[0, _NA_LANE])
    def _():
        x = xs_ref[...].astype(jnp.bfloat16)
        h = lax.dot_general(
            x, w1_ref[0].astype(jnp.bfloat16), (((1,), (1,)), ((), ())),
            preferred_element_type=jnp.float32,
        )
        h = jnp.maximum(h + b1_ref[0], 0.0).astype(jnp.bfloat16)
        r = lax.dot_general(
            h, w2_ref[0].astype(jnp.bfloat16), (((1,), (1,)), ((), ())),
            preferred_element_type=jnp.float32,
        )
        r = jnp.maximum(r + b2_ref[0], 0.0)
        out_ref[...] = r + 0.01 * r


def _tc_mlp(xs, W1, b1, W2, b2, meta):
    def xs_map(i, meta):
        return (jnp.minimum(i, meta[0, _NA_LANE] - 1), 0)

    grid_spec = pltpu.PrefetchScalarGridSpec(
        num_scalar_prefetch=1,
        grid=(NUM_BLOCKS,),
        in_specs=[
            pl.BlockSpec((BLK, HID), xs_map),
            pl.BlockSpec((1, HID, HID), lambda i, meta: (meta[0, i], 0, 0)),
            pl.BlockSpec((1, 1, HID), lambda i, meta: (meta[0, i], 0, 0)),
            pl.BlockSpec((1, HID, HID), lambda i, meta: (meta[0, i], 0, 0)),
            pl.BlockSpec((1, 1, HID), lambda i, meta: (meta[0, i], 0, 0)),
        ],
        out_specs=pl.BlockSpec((BLK, HID), xs_map),
    )
    return pl.pallas_call(
        _tc_mlp_body,
        grid_spec=grid_spec,
        out_shape=jax.ShapeDtypeStruct((PADDED, HID), jnp.float32),
    )(
        meta,
        xs,
        W1,
        b1.reshape(N_EXP, 1, HID),
        W2.astype(jnp.bfloat16),
        b2.reshape(N_EXP, 1, HID),
    )


def kernel(x, chosen_ops, W1, b1, W2, b2):
    dest, meta = _route_metadata(chosen_ops)
    xs = _sc_scatter_rows(x, dest)
    ys = _tc_mlp(xs, W1, b1, W2, b2, meta)
    out = _sc_gather_rows(ys, dest.reshape(TOK), TOK)
    return out


# speedup vs baseline: 1.1211x; 1.1211x over previous
import functools

import jax
import jax.numpy as jnp
from jax import lax
from jax.experimental import pallas as pl
from jax.experimental.pallas import tpu as pltpu
from jax.experimental.pallas import tpu_sc as plsc

N_EXP = 8
HID = 1024
TOK = 4096
BLK = 512
NUM_BLOCKS = TOK // BLK + N_EXP
PADDED = NUM_BLOCKS * BLK

NC = 2
NS = 16
NW = NC * NS


_ROWS = 32
_LANES = TOK // _ROWS
_NA_LANE = 120


def _route_body(e_ref, dest_ref, meta_ref):
    e = e_ref[...]
    li = lax.broadcasted_iota(jnp.int32, (_LANES, _LANES), 0)
    lj = lax.broadcasted_iota(jnp.int32, (_LANES, _LANES), 1)
    tri_incl = (li <= lj).astype(jnp.bfloat16)
    ri = lax.broadcasted_iota(jnp.int32, (_ROWS, _ROWS), 0)
    rj = lax.broadcasted_iota(jnp.int32, (_ROWS, _ROWS), 1)
    tri_strict = (rj < ri).astype(jnp.bfloat16)
    ones_col = jnp.ones((_LANES, 1), jnp.bfloat16)

    dest = jnp.zeros((_ROWS, _LANES), jnp.float32)
    offs = 0.0
    cumblk = []
    present = []
    for k in range(N_EXP):
        m = (e == k).astype(jnp.bfloat16)
        pfx = lax.dot_general(
            m, tri_incl, (((1,), (0,)), ((), ())),
            preferred_element_type=jnp.float32,
        )
        rowsum = lax.dot_general(
            m, ones_col, (((1,), (0,)), ((), ())),
            preferred_element_type=jnp.float32,
        )
        rowpfx = lax.dot_general(
            tri_strict, rowsum.astype(jnp.bfloat16), (((1,), (0,)), ((), ())),
            preferred_element_type=jnp.float32,
        )
        g = pfx + rowpfx
        count = jnp.sum(rowsum)
        padded = jnp.floor((count + (BLK - 1)) * (1.0 / BLK)) * BLK
        dest = dest + m.astype(jnp.float32) * (offs - 1.0 + g)
        offs = offs + padded
        cumblk.append(offs * (1.0 / BLK))
        present.append(count > 0.5)
    dest_ref[...] = dest.astype(jnp.int32)

    lane = lax.broadcasted_iota(jnp.int32, (8, _LANES), 1).astype(jnp.float32)
    be = jnp.zeros((8, _LANES), jnp.float32)
    for k in range(N_EXP):
        be = be + (lane >= cumblk[k]).astype(jnp.float32)
    be = jnp.minimum(be, float(N_EXP - 1))
    nxt = jnp.full((8, _LANES), float(N_EXP))
    for k in range(N_EXP - 1, -1, -1):
        nxt = jnp.where(jnp.logical_and(be < float(k), present[k]), float(k), nxt)
    grank = jnp.zeros((8, _LANES), jnp.float32)
    for k in range(N_EXP):
        grank = grank + jnp.where(
            jnp.logical_and(be > float(k), present[k]), 1.0, 0.0
        )
    row0 = jnp.where(lane == float(_NA_LANE), cumblk[N_EXP - 1], be)
    ridx = lax.broadcasted_iota(jnp.int32, (8, _LANES), 0)
    out = jnp.where(ridx == 1, nxt, row0)
    out = jnp.where(ridx == 2, grank, out)
    meta_ref[...] = out.astype(jnp.int32)


def _route_metadata(chosen_ops):
    e32 = chosen_ops.astype(jnp.int32).reshape(_ROWS, _LANES)
    dest, meta = pl.pallas_call(
        _route_body,
        out_shape=(
            jax.ShapeDtypeStruct((_ROWS, _LANES), jnp.int32),
            jax.ShapeDtypeStruct((8, _LANES), jnp.int32),
        ),
    )(e32)
    return dest, meta


def _sc_scatter_rows(x, dest):
    bpw = TOK // NW
    chunk = 32
    n_chunks = bpw // chunk
    dest3 = dest.reshape(NW, n_chunks, chunk)
    mesh = plsc.VectorSubcoreMesh(core_axis_name="c", subcore_axis_name="s")

    @functools.partial(
        pl.kernel,
        mesh=mesh,
        out_type=jax.ShapeDtypeStruct((PADDED, HID), jnp.float32),
        scratch_types=[
            pltpu.VMEM((n_chunks, chunk), jnp.int32),
            pltpu.VMEM((chunk, HID), jnp.float32),
            pltpu.VMEM((chunk, HID), jnp.float32),
            pltpu.SemaphoreType.DMA,
            pltpu.SemaphoreType.DMA,
        ],
    )
    def scatter_kernel(x_hbm, idx_hbm, out_hbm, idx_v, rows_a, rows_b, sem_a, sem_b):
        wid = lax.axis_index("s") * NC + lax.axis_index("c")
        base = wid * bpw
        pltpu.sync_copy(idx_hbm.at[wid], idx_v)
        bufs = (rows_a, rows_b)
        sems = (sem_a, sem_b)
        copies = [None, None]
        for c in range(n_chunks):
            if c >= 2:
                copies[c % 2].wait()
            pltpu.sync_copy(x_hbm.at[pl.ds(base + c * chunk, chunk)], bufs[c % 2])
            copies[c % 2] = pltpu.async_copy(
                bufs[c % 2], out_hbm.at[idx_v.at[c]], sems[c % 2]
            )
        copies[(n_chunks - 2) % 2].wait()
        copies[(n_chunks - 1) % 2].wait()

    return scatter_kernel(x, dest3)


def _sc_gather_rows(table, idx, n_rows):
    bpw = n_rows // NW
    chunk = 32
    n_chunks = bpw // chunk
    mesh = plsc.VectorSubcoreMesh(core_axis_name="c", subcore_axis_name="s")

    @functools.partial(
        pl.kernel,
        mesh=mesh,
        out_type=jax.ShapeDtypeStruct((n_rows, HID), jnp.float32),
        scratch_types=[
            pltpu.VMEM((bpw,), jnp.int32),
            pltpu.VMEM((chunk, HID), jnp.float32),
            pltpu.VMEM((chunk, HID), jnp.float32),
            pltpu.SemaphoreType.DMA,
            pltpu.SemaphoreType.DMA,
        ],
    )
    def gather_kernel(table_hbm, idx_hbm, out_hbm, idx_v, rows_a, rows_b, sem_a, sem_b):
        wid = lax.axis_index("s") * NC + lax.axis_index("c")
        base = wid * bpw
        pltpu.sync_copy(idx_hbm.at[pl.ds(base, bpw)], idx_v)
        bufs = (rows_a, rows_b)
        sems = (sem_a, sem_b)
        copies = [None, None]
        for c in range(n_chunks):
            copies[c % 2] = pltpu.async_copy(
                table_hbm.at[idx_v.at[pl.ds(c * chunk, chunk)]],
                bufs[c % 2],
                sems[c % 2],
            )
            if c >= 1:
                copies[(c - 1) % 2].wait()
                pltpu.sync_copy(
                    bufs[(c - 1) % 2],
                    out_hbm.at[pl.ds(base + (c - 1) * chunk, chunk)],
                )
        copies[(n_chunks - 1) % 2].wait()
        pltpu.sync_copy(
            bufs[(n_chunks - 1) % 2],
            out_hbm.at[pl.ds(base + (n_chunks - 1) * chunk, chunk)],
        )

    return gather_kernel(table, idx)


def _tc_mlp_body(
    meta_ref, xs_ref, w1_hbm, b1_ref, w2_hbm, b2_ref, out_ref, w1s, w2s, sems
):
    i = pl.program_id(0)
    active = i < meta_ref[0, _NA_LANE]
    cur = meta_ref[0, i]
    nxt = meta_ref[1, i]
    slot = meta_ref[2, i] % 2
    prev = meta_ref[0, jnp.maximum(i - 1, 0)]
    switch = jnp.logical_or(i == 0, prev != cur)

    @pl.when(jnp.logical_and(active, i == 0))
    def _():
        pltpu.make_async_copy(w1_hbm.at[cur], w1s.at[slot], sems.at[0, slot]).start()
        pltpu.make_async_copy(w2_hbm.at[cur], w2s.at[slot], sems.at[1, slot]).start()

    @pl.when(jnp.logical_and(active, jnp.logical_and(switch, nxt < N_EXP)))
    def _():
        oslot = 1 - slot
        pltpu.make_async_copy(w1_hbm.at[nxt], w1s.at[oslot], sems.at[0, oslot]).start()
        pltpu.make_async_copy(w2_hbm.at[nxt], w2s.at[oslot], sems.at[1, oslot]).start()

    @pl.when(jnp.logical_and(active, switch))
    def _():
        pltpu.make_async_copy(w1_hbm.at[cur], w1s.at[slot], sems.at[0, slot]).wait()
        pltpu.make_async_copy(w2_hbm.at[cur], w2s.at[slot], sems.at[1, slot]).wait()

    @pl.when(active)
    def _():
        x = xs_ref[...].astype(jnp.bfloat16)
        h = lax.dot_general(
            x, w1s[slot].astype(jnp.bfloat16), (((1,), (1,)), ((), ())),
            preferred_element_type=jnp.float32,
        )
        h = jnp.maximum(h + b1_ref[0], 0.0).astype(jnp.bfloat16)
        r = lax.dot_general(
            h, w2s[slot].astype(jnp.bfloat16), (((1,), (1,)), ((), ())),
            preferred_element_type=jnp.float32,
        )
        r = jnp.maximum(r + b2_ref[0], 0.0)
        out_ref[...] = r + 0.01 * r


def _tc_mlp(xs, W1, b1, W2, b2, meta):
    def xs_map(i, meta):
        return (jnp.minimum(i, meta[0, _NA_LANE] - 1), 0)

    grid_spec = pltpu.PrefetchScalarGridSpec(
        num_scalar_prefetch=1,
        grid=(NUM_BLOCKS,),
        in_specs=[
            pl.BlockSpec((BLK, HID), xs_map),
            pl.BlockSpec(memory_space=pl.ANY),
            pl.BlockSpec((1, 1, HID), lambda i, meta: (meta[0, i], 0, 0)),
            pl.BlockSpec(memory_space=pl.ANY),
            pl.BlockSpec((1, 1, HID), lambda i, meta: (meta[0, i], 0, 0)),
        ],
        out_specs=pl.BlockSpec((BLK, HID), xs_map),
        scratch_shapes=[
            pltpu.VMEM((2, HID, HID), jnp.float32),
            pltpu.VMEM((2, HID, HID), jnp.float32),
            pltpu.SemaphoreType.DMA((2, 2)),
        ],
    )
    return pl.pallas_call(
        _tc_mlp_body,
        grid_spec=grid_spec,
        out_shape=jax.ShapeDtypeStruct((PADDED, HID), jnp.float32),
    )(
        meta,
        xs,
        W1,
        b1.reshape(N_EXP, 1, HID),
        W2,
        b2.reshape(N_EXP, 1, HID),
    )


def kernel(x, chosen_ops, W1, b1, W2, b2):
    dest, meta = _route_metadata(chosen_ops)
    xs = _sc_scatter_rows(x, dest)
    ys = _tc_mlp(xs, W1, b1, W2, b2, meta)
    out = _sc_gather_rows(ys, dest.reshape(TOK), TOK)
    return out


# speedup vs baseline: 1.1288x; 1.0068x over previous
import functools

import jax
import jax.numpy as jnp
from jax import lax
from jax.experimental import pallas as pl
from jax.experimental.pallas import tpu as pltpu
from jax.experimental.pallas import tpu_sc as plsc

N_EXP = 8
HID = 1024
TOK = 4096
BLK = 256
NUM_BLOCKS = TOK // BLK + N_EXP
PADDED = NUM_BLOCKS * BLK

NC = 2
NS = 16
NW = NC * NS


_ROWS = 32
_LANES = TOK // _ROWS
_NA_LANE = 120


def _route_body(e_ref, dest_ref, meta_ref):
    e = e_ref[...]
    li = lax.broadcasted_iota(jnp.int32, (_LANES, _LANES), 0)
    lj = lax.broadcasted_iota(jnp.int32, (_LANES, _LANES), 1)
    tri_incl = (li <= lj).astype(jnp.bfloat16)
    ri = lax.broadcasted_iota(jnp.int32, (_ROWS, _ROWS), 0)
    rj = lax.broadcasted_iota(jnp.int32, (_ROWS, _ROWS), 1)
    tri_strict = (rj < ri).astype(jnp.bfloat16)
    ones_col = jnp.ones((_LANES, 1), jnp.bfloat16)

    dest = jnp.zeros((_ROWS, _LANES), jnp.float32)
    offs = 0.0
    cumblk = []
    present = []
    for k in range(N_EXP):
        m = (e == k).astype(jnp.bfloat16)
        pfx = lax.dot_general(
            m, tri_incl, (((1,), (0,)), ((), ())),
            preferred_element_type=jnp.float32,
        )
        rowsum = lax.dot_general(
            m, ones_col, (((1,), (0,)), ((), ())),
            preferred_element_type=jnp.float32,
        )
        rowpfx = lax.dot_general(
            tri_strict, rowsum.astype(jnp.bfloat16), (((1,), (0,)), ((), ())),
            preferred_element_type=jnp.float32,
        )
        g = pfx + rowpfx
        count = jnp.sum(rowsum)
        padded = jnp.floor((count + (BLK - 1)) * (1.0 / BLK)) * BLK
        dest = dest + m.astype(jnp.float32) * (offs - 1.0 + g)
        offs = offs + padded
        cumblk.append(offs * (1.0 / BLK))
        present.append(count > 0.5)
    dest_ref[...] = dest.astype(jnp.int32)

    lane = lax.broadcasted_iota(jnp.int32, (8, _LANES), 1).astype(jnp.float32)
    be = jnp.zeros((8, _LANES), jnp.float32)
    for k in range(N_EXP):
        be = be + (lane >= cumblk[k]).astype(jnp.float32)
    be = jnp.minimum(be, float(N_EXP - 1))
    nxt = jnp.full((8, _LANES), float(N_EXP))
    for k in range(N_EXP - 1, -1, -1):
        nxt = jnp.where(jnp.logical_and(be < float(k), present[k]), float(k), nxt)
    grank = jnp.zeros((8, _LANES), jnp.float32)
    for k in range(N_EXP):
        grank = grank + jnp.where(
            jnp.logical_and(be > float(k), present[k]), 1.0, 0.0
        )
    nn = jnp.full((8, _LANES), float(N_EXP))
    for k in range(N_EXP - 1, -1, -1):
        nn = jnp.where(jnp.logical_and(nxt < float(k), present[k]), float(k), nn)
    row0 = jnp.where(lane == float(_NA_LANE), cumblk[N_EXP - 1], be)
    ridx = lax.broadcasted_iota(jnp.int32, (8, _LANES), 0)
    out = jnp.where(ridx == 1, nxt, row0)
    out = jnp.where(ridx == 2, grank, out)
    out = jnp.where(ridx == 3, nn, out)
    meta_ref[...] = out.astype(jnp.int32)


def _route_metadata(chosen_ops):
    e32 = chosen_ops.astype(jnp.int32).reshape(_ROWS, _LANES)
    dest, meta = pl.pallas_call(
        _route_body,
        out_shape=(
            jax.ShapeDtypeStruct((_ROWS, _LANES), jnp.int32),
            jax.ShapeDtypeStruct((8, _LANES), jnp.int32),
        ),
    )(e32)
    return dest, meta


def _sc_scatter_rows(x, dest):
    bpw = TOK // NW
    chunk = 32
    n_chunks = bpw // chunk
    dest3 = dest.reshape(NW, n_chunks, chunk)
    mesh = plsc.VectorSubcoreMesh(core_axis_name="c", subcore_axis_name="s")

    @functools.partial(
        pl.kernel,
        mesh=mesh,
        out_type=jax.ShapeDtypeStruct((PADDED, HID), jnp.float32),
        scratch_types=[
            pltpu.VMEM((n_chunks, chunk), jnp.int32),
            pltpu.VMEM((chunk, HID), jnp.float32),
            pltpu.VMEM((chunk, HID), jnp.float32),
            pltpu.SemaphoreType.DMA,
            pltpu.SemaphoreType.DMA,
        ],
    )
    def scatter_kernel(x_hbm, idx_hbm, out_hbm, idx_v, rows_a, rows_b, sem_a, sem_b):
        wid = lax.axis_index("s") * NC + lax.axis_index("c")
        base = wid * bpw
        pltpu.sync_copy(idx_hbm.at[wid], idx_v)
        bufs = (rows_a, rows_b)
        sems = (sem_a, sem_b)
        copies = [None, None]
        for c in range(n_chunks):
            if c >= 2:
                copies[c % 2].wait()
            pltpu.sync_copy(x_hbm.at[pl.ds(base + c * chunk, chunk)], bufs[c % 2])
            copies[c % 2] = pltpu.async_copy(
                bufs[c % 2], out_hbm.at[idx_v.at[c]], sems[c % 2]
            )
        copies[(n_chunks - 2) % 2].wait()
        copies[(n_chunks - 1) % 2].wait()

    return scatter_kernel(x, dest3)


def _sc_gather_rows(table, idx, n_rows):
    bpw = n_rows // NW
    chunk = 32
    n_chunks = bpw // chunk
    mesh = plsc.VectorSubcoreMesh(core_axis_name="c", subcore_axis_name="s")

    @functools.partial(
        pl.kernel,
        mesh=mesh,
        out_type=jax.ShapeDtypeStruct((n_rows, HID), jnp.float32),
        scratch_types=[
            pltpu.VMEM((bpw,), jnp.int32),
            pltpu.VMEM((chunk, HID), jnp.float32),
            pltpu.VMEM((chunk, HID), jnp.float32),
            pltpu.SemaphoreType.DMA,
            pltpu.SemaphoreType.DMA,
        ],
    )
    def gather_kernel(table_hbm, idx_hbm, out_hbm, idx_v, rows_a, rows_b, sem_a, sem_b):
        wid = lax.axis_index("s") * NC + lax.axis_index("c")
        base = wid * bpw
        pltpu.sync_copy(idx_hbm.at[pl.ds(base, bpw)], idx_v)
        bufs = (rows_a, rows_b)
        sems = (sem_a, sem_b)
        copies = [None, None]
        for c in range(n_chunks):
            copies[c % 2] = pltpu.async_copy(
                table_hbm.at[idx_v.at[pl.ds(c * chunk, chunk)]],
                bufs[c % 2],
                sems[c % 2],
            )
            if c >= 1:
                copies[(c - 1) % 2].wait()
                pltpu.sync_copy(
                    bufs[(c - 1) % 2],
                    out_hbm.at[pl.ds(base + (c - 1) * chunk, chunk)],
                )
        copies[(n_chunks - 1) % 2].wait()
        pltpu.sync_copy(
            bufs[(n_chunks - 1) % 2],
            out_hbm.at[pl.ds(base + (n_chunks - 1) * chunk, chunk)],
        )

    return gather_kernel(table, idx)


def _tc_mlp_body(
    meta_ref, xs_ref, w1_hbm, b1_ref, w2_hbm, b2_ref, out_ref, w1s, w2s, sems
):
    i = pl.program_id(0)
    active = i < meta_ref[0, _NA_LANE]
    cur = meta_ref[0, i]
    nxt = meta_ref[1, i]
    grank = meta_ref[2, i]
    slot = grank % 2
    prev = meta_ref[0, jnp.maximum(i - 1, 0)]
    switch = jnp.logical_or(i == 0, prev != cur)

    @pl.when(jnp.logical_and(active, i == 0))
    def _():
        pltpu.make_async_copy(w1_hbm.at[cur], w1s.at[slot], sems.at[0, slot]).start()
        pltpu.make_async_copy(w2_hbm.at[cur], w2s.at[slot], sems.at[1, slot]).start()

    @pl.when(jnp.logical_and(active, jnp.logical_and(switch, nxt < N_EXP)))
    def _():
        oslot = 1 - slot
        pltpu.make_async_copy(w1_hbm.at[nxt], w1s.at[oslot], sems.at[0, oslot]).start()
        pltpu.make_async_copy(w2_hbm.at[nxt], w2s.at[oslot], sems.at[1, oslot]).start()

    @pl.when(jnp.logical_and(active, switch))
    def _():
        pltpu.make_async_copy(w1_hbm.at[cur], w1s.at[slot], sems.at[0, slot]).wait()
        pltpu.make_async_copy(w2_hbm.at[cur], w2s.at[slot], sems.at[1, slot]).wait()

    @pl.when(active)
    def _():
        x = xs_ref[...].astype(jnp.bfloat16)
        h = lax.dot_general(
            x, w1s[slot].astype(jnp.bfloat16), (((1,), (1,)), ((), ())),
            preferred_element_type=jnp.float32,
        )
        h = jnp.maximum(h + b1_ref[0], 0.0).astype(jnp.bfloat16)
        r = lax.dot_general(
            h, w2s[slot].astype(jnp.bfloat16), (((1,), (1,)), ((), ())),
            preferred_element_type=jnp.float32,
        )
        r = jnp.maximum(r + b2_ref[0], 0.0)
        out_ref[...] = r + 0.01 * r


def _tc_mlp(xs, W1, b1, W2, b2, meta):
    def xs_map(i, meta):
        return (jnp.minimum(i, meta[0, _NA_LANE] - 1), 0)

    grid_spec = pltpu.PrefetchScalarGridSpec(
        num_scalar_prefetch=1,
        grid=(NUM_BLOCKS,),
        in_specs=[
            pl.BlockSpec((BLK, HID), xs_map),
            pl.BlockSpec(memory_space=pl.ANY),
            pl.BlockSpec((1, 1, HID), lambda i, meta: (meta[0, i], 0, 0)),
            pl.BlockSpec(memory_space=pl.ANY),
            pl.BlockSpec((1, 1, HID), lambda i, meta: (meta[0, i], 0, 0)),
        ],
        out_specs=pl.BlockSpec((BLK, HID), xs_map),
        scratch_shapes=[
            pltpu.VMEM((2, HID, HID), jnp.float32),
            pltpu.VMEM((2, HID, HID), jnp.float32),
            pltpu.SemaphoreType.DMA((2, 2)),
        ],
    )
    return pl.pallas_call(
        _tc_mlp_body,
        grid_spec=grid_spec,
        out_shape=jax.ShapeDtypeStruct((PADDED, HID), jnp.float32),
    )(
        meta,
        xs,
        W1,
        b1.reshape(N_EXP, 1, HID),
        W2,
        b2.reshape(N_EXP, 1, HID),
    )


def kernel(x, chosen_ops, W1, b1, W2, b2):
    dest, meta = _route_metadata(chosen_ops)
    xs = _sc_scatter_rows(x, dest)
    ys = _tc_mlp(xs, W1, b1, W2, b2, meta)
    out = _sc_gather_rows(ys, dest.reshape(TOK), TOK)
    return out
